# baseline (speedup 1.0000x reference)
import functools

import jax
import jax.numpy as jnp
from jax.experimental import pallas as pl
from jax.experimental.pallas import tpu as pltpu

_ALPHA = 0.2
_WDT = jnp.bfloat16
_BNS, _BR, _BUVA = 0, 4, 8


def _precompute(h, fh, x, wf_ref, ws_ref, wt_ref, cp_ref, t_t_ref):
    wh = jnp.dot(x, wf_ref[...], preferred_element_type=jnp.float32)
    s = jnp.dot(x, ws_ref[...], preferred_element_type=jnp.float32)
    t = jnp.dot(x, wt_ref[...], preferred_element_type=jnp.float32)
    u = jnp.exp(t)
    v = jnp.exp(_ALPHA * t)
    c = lambda a: a.astype(_WDT)
    wa, wc = 2 * fh + 2, fh + 1
    bv = _BUVA + h * wa
    for k in range(h):
        whk = wh[:, k * fh:(k + 1) * fh]
        uk = u[:, k:k + 1]
        vk = v[:, k:k + 1]
        o = _BUVA + k * wa
        cp_ref[:, o:o + fh] = c(whk * uk)
        cp_ref[:, o + fh:o + fh + 1] = c(uk)
        cp_ref[:, o + fh + 1:o + 2 * fh + 1] = c(whk * vk)
        cp_ref[:, o + 2 * fh + 1:o + wa] = c(vk)
        cp_ref[:, bv + k * wc:bv + k * wc + fh] = c(whk * vk)
        cp_ref[:, bv + k * wc + fh:bv + (k + 1) * wc] = c(vk)
    cp_ref[:, _BNS:_BNS + h] = c(-s)
    cp_ref[:, _BR:_BR + h] = c(jnp.exp((_ALPHA - 1.0) * s))
    t_t_ref[:h, :] = c(t.T)


def _pre0_kernel(h, fh, x_ref, wf_ref, ws_ref, wt_ref, cp_ref, t_t_ref):
    _precompute(h, fh, x_ref[...], wf_ref, ws_ref, wt_ref, cp_ref, t_t_ref)


def _attend(h, fh, bi, i, m, cp_ref, t_t_ref):
    sl = pl.ds(i * bi, bi)
    wa, wc = 2 * fh + 2, fh + 1
    bv = _BUVA + h * wa
    cm = jnp.dot(m, cp_ref[:, bv:bv + h * wc],
                 preferred_element_type=jnp.float32)
    outs = []
    for k in range(h):
        ns_c = cp_ref[sl, _BNS + k:_BNS + k + 1]
        mpos = jnp.where(t_t_ref[k:k + 1, :] > ns_c, m,
                         jnp.zeros((), _WDT))
        o = _BUVA + k * wa
        a = jnp.dot(mpos, cp_ref[:, o:o + wa],
                    preferred_element_type=jnp.float32)
        r_c = cp_ref[sl, _BR + k:_BR + k + 1].astype(jnp.float32)
        nd = a[:, :wc] + r_c * (cm[:, k * wc:(k + 1) * wc]
                                - a[:, wc:])
        outs.append(nd[:, :fh] / nd[:, fh:fh + 1])
    return outs


def _mega_kernel(layers, bi,
                 adj_ref, cp0_ref, tt0_ref, wf_refs, ws_refs, wt_refs,
                 out_ref, mask_ref, x_ref, cp_ref, t_t_ref):
    l = pl.program_id(0)
    i = pl.program_id(1)

    for lc in range(len(layers)):
        h, fh, fin = layers[lc]
        last = lc == len(layers) - 1

        if lc > 0:
            @pl.when(jnp.logical_and(l == lc, i == 0))
            def _(lc=lc, h=h, fh=fh, fin=fin):
                _precompute(h, fh, x_ref[:, :fin].astype(jnp.float32),
                            wf_refs[lc - 1],
                            ws_refs[lc - 1], wt_refs[lc - 1], cp_ref, t_t_ref)

        @pl.when(l == lc)
        def _(lc=lc, h=h, fh=fh, last=last):
            sl = pl.ds(i * bi, bi)
            if lc == 0:
                n = mask_ref.shape[1]
                for cs in range(0, n, n // 4):
                    mask_ref[sl, cs:cs + n // 4] = (
                        adj_ref[:, cs:cs + n // 4].astype(_WDT))
                m = mask_ref[sl, :]
                outs = _attend(h, fh, bi, i, m, cp0_ref, tt0_ref)
            else:
                m = mask_ref[sl, :]
                outs = _attend(h, fh, bi, i, m, cp_ref, t_t_ref)
            if not last:
                o = jnp.concatenate(outs, axis=1)
                x_ref[sl, :h * fh] = jnp.where(
                    o > 0, o, jnp.exp(o) - 1.0).astype(_WDT)
            else:
                o = outs[0]
                for x in outs[1:]:
                    o = o + x
                o = jnp.maximum(o * (1.0 / h), 0.0)
                part = jnp.sum(o, axis=0, keepdims=True)

                @pl.when(i == 0)
                def _():
                    out_ref[...] = jnp.zeros_like(out_ref)

                out_ref[...] += part


def kernel(node_features, adj_mat,
           W1, a1_src, a1_dst,
           W2, a2_src, a2_dst,
           W3, a3_src, a3_dst,
           W4, a4_src, a4_dst):
    n = node_features.shape[0]
    bi = min(128, n)
    nb = n // bi
    params = ((W1, a1_src, a1_dst), (W2, a2_src, a2_dst),
              (W3, a3_src, a3_dst), (W4, a4_src, a4_dst))
    layers = tuple((w.shape[1], w.shape[2], w.shape[0]) for w, _, _ in params)
    wfs, wss, wts = [], [], []
    for w, a_s, a_d in params:
        fin, h, fh = w.shape
        wf = w.reshape(fin, h * fh)
        wfs.append(wf)
        eye = jnp.eye(h, dtype=w.dtype)
        bd_s = (a_s[:, :, None] * eye[:, None, :]).reshape(h * fh, h)
        bd_d = (a_d[:, :, None] * eye[:, None, :]).reshape(h * fh, h)
        wss.append(wf @ bd_s)
        wts.append(wf @ bd_d)

    cols = lambda h, fh: _BUVA + h * (3 * fh + 3)
    h0, fh0, _ = layers[0]
    c0 = -(-cols(h0, fh0) // 128) * 128
    pre0 = pl.pallas_call(
        functools.partial(_pre0_kernel, h0, fh0),
        out_shape=(
            jax.ShapeDtypeStruct((n, c0), _WDT),
            jax.ShapeDtypeStruct((8, n), _WDT),
        ),
    )
    cp0, tt0 = pre0(node_features, wfs[0], wss[0], wts[0])

    full = lambda a: pl.BlockSpec(a.shape, lambda l, i: (0,) * a.ndim)
    max_cols = -(-max(cols(h, fh) for h, fh, _ in layers[1:]) // 128) * 128
    max_xf = max(h * fh for h, fh, _ in layers[:-1])
    fh_last = layers[-1][1]
    out = pl.pallas_call(
        functools.partial(_mega_kernel, layers, bi),
        grid=(len(layers), nb),
        compiler_params=pltpu.CompilerParams(vmem_limit_bytes=67_043_328),
        in_specs=[
            pl.BlockSpec((bi, n), lambda l, i: (jnp.where(l == 0, i, 0), 0)),
            full(cp0),
            full(tt0),
            [full(w) for w in wfs[1:]],
            [full(w) for w in wss[1:]],
            [full(w) for w in wts[1:]],
        ],
        out_specs=pl.BlockSpec((1, fh_last), lambda l, i: (0, 0)),
        out_shape=jax.ShapeDtypeStruct((1, fh_last), jnp.float32),
        scratch_shapes=[
            pltpu.VMEM((n, n), _WDT),
            pltpu.VMEM((n, max_xf), _WDT),
            pltpu.VMEM((n, max_cols), _WDT),
            pltpu.VMEM((8, n), _WDT),
        ],
    )(adj_mat, cp0, tt0, wfs[1:], wss[1:], wts[1:])
    return out.reshape(-1)


# speedup vs baseline: 1.7632x; 1.7632x over previous
import functools

import jax
import jax.numpy as jnp
from jax.experimental import pallas as pl
from jax.experimental.pallas import tpu as pltpu

_ALPHA = 0.2
_WDT = jnp.bfloat16
_BNS, _BR, _BUVA = 0, 4, 8


def _precompute(h, fh, x, wf_ref, ws_ref, wt_ref, cp_ref, t_t_ref):
    wh = jnp.dot(x, wf_ref[...], preferred_element_type=jnp.float32)
    s = jnp.dot(x, ws_ref[...], preferred_element_type=jnp.float32)
    t = jnp.dot(x, wt_ref[...], preferred_element_type=jnp.float32)
    u = jnp.exp(t)
    v = jnp.exp(_ALPHA * t)
    c = lambda a: a.astype(_WDT)
    wa, wc = 2 * fh + 2, fh + 1
    bv = _BUVA + h * wa
    for k in range(h):
        whk = wh[:, k * fh:(k + 1) * fh]
        uk = u[:, k:k + 1]
        vk = v[:, k:k + 1]
        o = _BUVA + k * wa
        cp_ref[:, o:o + fh] = c(whk * uk)
        cp_ref[:, o + fh:o + fh + 1] = c(uk)
        cp_ref[:, o + fh + 1:o + 2 * fh + 1] = c(whk * vk)
        cp_ref[:, o + 2 * fh + 1:o + wa] = c(vk)
        cp_ref[:, bv + k * wc:bv + k * wc + fh] = c(whk * vk)
        cp_ref[:, bv + k * wc + fh:bv + (k + 1) * wc] = c(vk)
    cp_ref[:, _BNS:_BNS + h] = c(-s)
    cp_ref[:, _BR:_BR + h] = c(jnp.exp((_ALPHA - 1.0) * s))
    t_t_ref[:h, :] = c(t.T)


def _pre0_kernel(h, fh, x_ref, wf_ref, ws_ref, wt_ref, cp_ref, t_t_ref):
    _precompute(h, fh, x_ref[...], wf_ref, ws_ref, wt_ref, cp_ref, t_t_ref)


def _attend(h, fh, bi, i, m, cp_ref, t_t_ref):
    sl = pl.ds(i * bi, bi)
    wa, wc = 2 * fh + 2, fh + 1
    bv = _BUVA + h * wa
    cm = jnp.dot(m, cp_ref[:, bv:bv + h * wc],
                 preferred_element_type=jnp.float32)
    outs = []
    for k in range(h):
        ns_c = cp_ref[sl, _BNS + k:_BNS + k + 1]
        mpos = jnp.where(t_t_ref[k:k + 1, :] > ns_c, m,
                         jnp.zeros((), _WDT))
        o = _BUVA + k * wa
        a = jnp.dot(mpos, cp_ref[:, o:o + wa],
                    preferred_element_type=jnp.float32)
        r_c = cp_ref[sl, _BR + k:_BR + k + 1].astype(jnp.float32)
        nd = a[:, :wc] + r_c * (cm[:, k * wc:(k + 1) * wc]
                                - a[:, wc:])
        outs.append(nd[:, :fh] / nd[:, fh:fh + 1])
    return outs


def _mega_kernel(layers, bi,
                 adj_ref, cp0_ref, tt0_ref, wf_refs, ws_refs, wt_refs,
                 out_ref, mask_ref, x_ref, cp_ref, t_t_ref):
    l = pl.program_id(0)
    i = pl.program_id(1)

    for lc in range(len(layers)):
        h, fh, fin = layers[lc]
        last = lc == len(layers) - 1

        if lc > 0:
            @pl.when(jnp.logical_and(l == lc, i == 0))
            def _(lc=lc, h=h, fh=fh, fin=fin):
                _precompute(h, fh, x_ref[:, :fin].astype(jnp.float32),
                            wf_refs[lc - 1],
                            ws_refs[lc - 1], wt_refs[lc - 1], cp_ref, t_t_ref)

        @pl.when(l == lc)
        def _(lc=lc, h=h, fh=fh, last=last):
            sl = pl.ds(i * bi, bi)
            if lc == 0:
                mask_ref[sl, :] = adj_ref[...].astype(jnp.int8)
                m = mask_ref[sl, :].astype(_WDT)
                outs = _attend(h, fh, bi, i, m, cp0_ref, tt0_ref)
            else:
                m = mask_ref[sl, :].astype(_WDT)
                outs = _attend(h, fh, bi, i, m, cp_ref, t_t_ref)
            if not last:
                o = jnp.concatenate(outs, axis=1)
                x_ref[sl, :h * fh] = jnp.where(
                    o > 0, o, jnp.exp(o) - 1.0).astype(_WDT)
            else:
                o = outs[0]
                for x in outs[1:]:
                    o = o + x
                o = jnp.maximum(o * (1.0 / h), 0.0)
                part = jnp.sum(o, axis=0, keepdims=True)

                @pl.when(i == 0)
                def _():
                    out_ref[...] = jnp.zeros_like(out_ref)

                out_ref[...] += part


def kernel(node_features, adj_mat,
           W1, a1_src, a1_dst,
           W2, a2_src, a2_dst,
           W3, a3_src, a3_dst,
           W4, a4_src, a4_dst):
    n = node_features.shape[0]
    bi = min(256, n)
    nb = n // bi
    params = ((W1, a1_src, a1_dst), (W2, a2_src, a2_dst),
              (W3, a3_src, a3_dst), (W4, a4_src, a4_dst))
    layers = tuple((w.shape[1], w.shape[2], w.shape[0]) for w, _, _ in params)
    wfs, wss, wts = [], [], []
    for w, a_s, a_d in params:
        fin, h, fh = w.shape
        wf = w.reshape(fin, h * fh)
        wfs.append(wf)
        eye = jnp.eye(h, dtype=w.dtype)
        bd_s = (a_s[:, :, None] * eye[:, None, :]).reshape(h * fh, h)
        bd_d = (a_d[:, :, None] * eye[:, None, :]).reshape(h * fh, h)
        wss.append(wf @ bd_s)
        wts.append(wf @ bd_d)

    cols = lambda h, fh: _BUVA + h * (3 * fh + 3)
    h0, fh0, _ = layers[0]
    c0 = -(-cols(h0, fh0) // 128) * 128
    pre0 = pl.pallas_call(
        functools.partial(_pre0_kernel, h0, fh0),
        out_shape=(
            jax.ShapeDtypeStruct((n, c0), _WDT),
            jax.ShapeDtypeStruct((8, n), _WDT),
        ),
    )
    cp0, tt0 = pre0(node_features, wfs[0], wss[0], wts[0])

    full = lambda a: pl.BlockSpec(a.shape, lambda l, i: (0,) * a.ndim)
    max_cols = -(-max(cols(h, fh) for h, fh, _ in layers[1:]) // 128) * 128
    max_xf = max(h * fh for h, fh, _ in layers[:-1])
    fh_last = layers[-1][1]
    out = pl.pallas_call(
        functools.partial(_mega_kernel, layers, bi),
        grid=(len(layers), nb),
        compiler_params=pltpu.CompilerParams(vmem_limit_bytes=67_043_328),
        in_specs=[
            pl.BlockSpec((bi, n), lambda l, i: (jnp.where(l == 0, i, 0), 0)),
            full(cp0),
            full(tt0),
            [full(w) for w in wfs[1:]],
            [full(w) for w in wss[1:]],
            [full(w) for w in wts[1:]],
        ],
        out_specs=pl.BlockSpec((1, fh_last), lambda l, i: (0, 0)),
        out_shape=jax.ShapeDtypeStruct((1, fh_last), jnp.float32),
        scratch_shapes=[
            pltpu.VMEM((n, n), jnp.int8),
            pltpu.VMEM((n, max_xf), _WDT),
            pltpu.VMEM((n, max_cols), _WDT),
            pltpu.VMEM((8, n), _WDT),
        ],
    )(adj_mat, cp0, tt0, wfs[1:], wss[1:], wts[1:])
    return out.reshape(-1)


# speedup vs baseline: 20.1896x; 11.4502x over previous
import functools

import jax
import jax.numpy as jnp
from jax.experimental import pallas as pl
from jax.experimental.pallas import tpu as pltpu

_ALPHA = 0.2
_WDT = jnp.bfloat16


def _precompute(lc, layers, x, wf_ref, ws_ref, wt_ref,
                ns_ref, p_ref, q_ref, t_t_ref, u_t_ref, v_t_ref, whe_ref):
    h, fh, fin = layers[lc]
    n = x.shape[0]
    wh = jnp.dot(x, wf_ref[...], preferred_element_type=jnp.float32)
    s = jnp.dot(x, ws_ref[...], preferred_element_type=jnp.float32)
    t = jnp.dot(x, wt_ref[...], preferred_element_type=jnp.float32)
    ns_ref[:, :h] = (-s).astype(_WDT)
    p_ref[:, :h] = jnp.exp(s).astype(_WDT)
    q_ref[:, :h] = jnp.exp(_ALPHA * s).astype(_WDT)
    t_t = t.T
    t_t_ref[:h, :] = t_t.astype(_WDT)
    u_t_ref[:h, :] = jnp.exp(t_t).astype(_WDT)
    v_t_ref[:h, :] = jnp.exp(_ALPHA * t_t).astype(_WDT)
    ones = jnp.ones((n, 1), jnp.float32)
    whe = jnp.concatenate(
        [jnp.concatenate([wh[:, i * fh:(i + 1) * fh], ones], axis=1)
         for i in range(h)], axis=1)
    whe_ref[:, :h * (fh + 1)] = whe.astype(_WDT)


def _attend(lc, layers, bi, i, m, ns_ref, p_ref, q_ref,
            t_t_ref, u_t_ref, v_t_ref, whe_ref):
    h, fh, _ = layers[lc]
    sl = pl.ds(i * bi, bi)
    outs = []
    for k in range(h):
        ns_c = ns_ref[sl, k:k + 1]
        pos = t_t_ref[k:k + 1, :] > ns_c
        w = jnp.where(pos,
                      p_ref[sl, k:k + 1] * u_t_ref[k:k + 1, :],
                      q_ref[sl, k:k + 1] * v_t_ref[k:k + 1, :])
        w = w * m
        nd = jnp.dot(w, whe_ref[:, k * (fh + 1):(k + 1) * (fh + 1)],
                     preferred_element_type=jnp.float32)
        outs.append(nd[:, :fh] / nd[:, fh:fh + 1])
    return outs


def _mega_kernel(layers, bi,
                 adj_ref, x0_ref, wf_refs, ws_refs, wt_refs, out_ref,
                 mask_ref, x_ref, ns_ref, p_ref, q_ref,
                 t_t_ref, u_t_ref, v_t_ref, whe_ref):
    l = pl.program_id(0)
    i = pl.program_id(1)

    for lc in range(len(layers)):
        h, fh, fin = layers[lc]
        last = lc == len(layers) - 1

        @pl.when(jnp.logical_and(l == lc, i == 0))
        def _(lc=lc, fin=fin):
            x = x0_ref[...] if lc == 0 else x_ref[:, :fin]
            _precompute(lc, layers, x, wf_refs[lc], ws_refs[lc], wt_refs[lc],
                        ns_ref, p_ref, q_ref, t_t_ref, u_t_ref, v_t_ref,
                        whe_ref)

        @pl.when(l == lc)
        def _(lc=lc, h=h, fh=fh, last=last):
            sl = pl.ds(i * bi, bi)
            if lc == 0:
                m = adj_ref[...].astype(_WDT)
                mask_ref[sl, :] = m
            else:
                m = mask_ref[sl, :]
            outs = _attend(lc, layers, bi, i, m, ns_ref, p_ref, q_ref,
                           t_t_ref, u_t_ref, v_t_ref, whe_ref)
            if not last:
                o = jnp.concatenate(outs, axis=1)
                x_ref[sl, :h * fh] = jnp.where(o > 0, o, jnp.exp(o) - 1.0)
            else:
                o = outs[0]
                for x in outs[1:]:
                    o = o + x
                o = jnp.maximum(o * (1.0 / h), 0.0)
                part = jnp.sum(o, axis=0, keepdims=True)

                @pl.when(i == 0)
                def _():
                    out_ref[...] = jnp.zeros_like(out_ref)

                out_ref[...] += part


def kernel(node_features, adj_mat,
           W1, a1_src, a1_dst,
           W2, a2_src, a2_dst,
           W3, a3_src, a3_dst,
           W4, a4_src, a4_dst):
    n = node_features.shape[0]
    bi = min(256, n)
    nb = n // bi
    params = ((W1, a1_src, a1_dst), (W2, a2_src, a2_dst),
              (W3, a3_src, a3_dst), (W4, a4_src, a4_dst))
    layers = tuple((w.shape[1], w.shape[2], w.shape[0]) for w, _, _ in params)
    wfs, wss, wts = [], [], []
    for w, a_s, a_d in params:
        fin, h, fh = w.shape
        wf = w.reshape(fin, h * fh)
        wfs.append(wf)
        eye = jnp.eye(h, dtype=w.dtype)
        bd_s = (a_s[:, :, None] * eye[:, None, :]).reshape(h * fh, h)
        bd_d = (a_d[:, :, None] * eye[:, None, :]).reshape(h * fh, h)
        wss.append(wf @ bd_s)
        wts.append(wf @ bd_d)

    full = lambda a: pl.BlockSpec(a.shape, lambda l, i: (0,) * a.ndim)
    max_whe = max(h * (fh + 1) for h, fh, _ in layers)
    max_xf = max(h * fh for h, fh, _ in layers[:-1])
    fh_last = layers[-1][1]
    out = pl.pallas_call(
        functools.partial(_mega_kernel, layers, bi),
        grid=(len(layers), nb),
        in_specs=[
            pl.BlockSpec((bi, n), lambda l, i: (jnp.where(l == 0, i, 0), 0)),
            full(node_features),
            [full(w) for w in wfs],
            [full(w) for w in wss],
            [full(w) for w in wts],
        ],
        out_specs=pl.BlockSpec((1, fh_last), lambda l, i: (0, 0)),
        out_shape=jax.ShapeDtypeStruct((1, fh_last), jnp.float32),
        scratch_shapes=[
            pltpu.VMEM((n, n), _WDT),
            pltpu.VMEM((n, max_xf), jnp.float32),
            pltpu.VMEM((n, 8), _WDT),
            pltpu.VMEM((n, 8), _WDT),
            pltpu.VMEM((n, 8), _WDT),
            pltpu.VMEM((8, n), _WDT),
            pltpu.VMEM((8, n), _WDT),
            pltpu.VMEM((8, n), _WDT),
            pltpu.VMEM((n, max_whe), _WDT),
        ],
    )(adj_mat, node_features, wfs, wss, wts)
    return out.reshape(-1)


# speedup vs baseline: 22.9110x; 1.1348x over previous
import functools

import jax
import jax.numpy as jnp
from jax.experimental import pallas as pl
from jax.experimental.pallas import tpu as pltpu

_ALPHA = 0.2
_WDT = jnp.bfloat16


def _precompute(lc, layers, x, wf_ref, ws_ref, wt_ref,
                ns_ref, r_ref, t_t_ref, u_t_ref, v_t_ref, whe_ref):
    h, fh, fin = layers[lc]
    n = x.shape[0]
    wh = jnp.dot(x, wf_ref[...], preferred_element_type=jnp.float32)
    s = jnp.dot(x, ws_ref[...], preferred_element_type=jnp.float32)
    t = jnp.dot(x, wt_ref[...], preferred_element_type=jnp.float32)
    ns_ref[:, :h] = (-s).astype(_WDT)
    r_ref[:, :h] = jnp.exp((_ALPHA - 1.0) * s).astype(_WDT)
    t_t = t.T
    t_t_ref[:h, :] = t_t.astype(_WDT)
    u_t_ref[:h, :] = jnp.exp(t_t).astype(_WDT)
    v_t_ref[:h, :] = jnp.exp(_ALPHA * t_t).astype(_WDT)
    ones = jnp.ones((n, 1), jnp.float32)
    whe = jnp.concatenate(
        [jnp.concatenate([wh[:, i * fh:(i + 1) * fh], ones], axis=1)
         for i in range(h)], axis=1)
    whe_ref[:, :h * (fh + 1)] = whe.astype(_WDT)


def _attend(lc, layers, bi, i, m, ns_ref, r_ref,
            t_t_ref, u_t_ref, v_t_ref, whe_ref):
    h, fh, _ = layers[lc]
    sl = pl.ds(i * bi, bi)
    outs = []
    for k in range(h):
        ns_c = ns_ref[sl, k:k + 1]
        pos = t_t_ref[k:k + 1, :] > ns_c
        w = jnp.where(pos,
                      u_t_ref[k:k + 1, :],
                      r_ref[sl, k:k + 1] * v_t_ref[k:k + 1, :])
        w = w * m
        nd = jnp.dot(w, whe_ref[:, k * (fh + 1):(k + 1) * (fh + 1)],
                     preferred_element_type=jnp.float32)
        outs.append(nd[:, :fh] / nd[:, fh:fh + 1])
    return outs


def _mega_kernel(layers, bi,
                 adj_ref, x0_ref, wf_refs, ws_refs, wt_refs, out_ref,
                 mask_ref, x_ref, ns_ref, r_ref,
                 t_t_ref, u_t_ref, v_t_ref, whe_ref):
    l = pl.program_id(0)
    i = pl.program_id(1)

    for lc in range(len(layers)):
        h, fh, fin = layers[lc]
        last = lc == len(layers) - 1

        @pl.when(jnp.logical_and(l == lc, i == 0))
        def _(lc=lc, fin=fin):
            x = x0_ref[...] if lc == 0 else x_ref[:, :fin]
            _precompute(lc, layers, x, wf_refs[lc], ws_refs[lc], wt_refs[lc],
                        ns_ref, r_ref, t_t_ref, u_t_ref, v_t_ref, whe_ref)

        @pl.when(l == lc)
        def _(lc=lc, h=h, fh=fh, last=last):
            sl = pl.ds(i * bi, bi)
            if lc == 0:
                m = adj_ref[...].astype(_WDT)
                mask_ref[sl, :] = m
            else:
                m = mask_ref[sl, :]
            outs = _attend(lc, layers, bi, i, m, ns_ref, r_ref,
                           t_t_ref, u_t_ref, v_t_ref, whe_ref)
            if not last:
                o = jnp.concatenate(outs, axis=1)
                x_ref[sl, :h * fh] = jnp.where(o > 0, o, jnp.exp(o) - 1.0)
            else:
                o = outs[0]
                for x in outs[1:]:
                    o = o + x
                o = jnp.maximum(o * (1.0 / h), 0.0)
                part = jnp.sum(o, axis=0, keepdims=True)

                @pl.when(i == 0)
                def _():
                    out_ref[...] = jnp.zeros_like(out_ref)

                out_ref[...] += part


def kernel(node_features, adj_mat,
           W1, a1_src, a1_dst,
           W2, a2_src, a2_dst,
           W3, a3_src, a3_dst,
           W4, a4_src, a4_dst):
    n = node_features.shape[0]
    bi = min(256, n)
    nb = n // bi
    params = ((W1, a1_src, a1_dst), (W2, a2_src, a2_dst),
              (W3, a3_src, a3_dst), (W4, a4_src, a4_dst))
    layers = tuple((w.shape[1], w.shape[2], w.shape[0]) for w, _, _ in params)
    wfs, wss, wts = [], [], []
    for w, a_s, a_d in params:
        fin, h, fh = w.shape
        wf = w.reshape(fin, h * fh)
        wfs.append(wf)
        eye = jnp.eye(h, dtype=w.dtype)
        bd_s = (a_s[:, :, None] * eye[:, None, :]).reshape(h * fh, h)
        bd_d = (a_d[:, :, None] * eye[:, None, :]).reshape(h * fh, h)
        wss.append(wf @ bd_s)
        wts.append(wf @ bd_d)

    full = lambda a: pl.BlockSpec(a.shape, lambda l, i: (0,) * a.ndim)
    max_whe = max(h * (fh + 1) for h, fh, _ in layers)
    max_xf = max(h * fh for h, fh, _ in layers[:-1])
    fh_last = layers[-1][1]
    out = pl.pallas_call(
        functools.partial(_mega_kernel, layers, bi),
        grid=(len(layers), nb),
        in_specs=[
            pl.BlockSpec((bi, n), lambda l, i: (jnp.where(l == 0, i, 0), 0)),
            full(node_features),
            [full(w) for w in wfs],
            [full(w) for w in wss],
            [full(w) for w in wts],
        ],
        out_specs=pl.BlockSpec((1, fh_last), lambda l, i: (0, 0)),
        out_shape=jax.ShapeDtypeStruct((1, fh_last), jnp.float32),
        scratch_shapes=[
            pltpu.VMEM((n, n), _WDT),
            pltpu.VMEM((n, max_xf), jnp.float32),
            pltpu.VMEM((n, 8), _WDT),
            pltpu.VMEM((n, 8), _WDT),
            pltpu.VMEM((8, n), _WDT),
            pltpu.VMEM((8, n), _WDT),
            pltpu.VMEM((8, n), _WDT),
            pltpu.VMEM((n, max_whe), _WDT),
        ],
    )(adj_mat, node_features, wfs, wss, wts)
    return out.reshape(-1)
